# Initial kernel scaffold; baseline (speedup 1.0000x reference)
import jax
import jax.numpy as jnp
from jax.experimental import pallas as pl


def kernel(features, edge_index, W1, b1, e1a_W, e1a_b, e1a_g, e1a_be, e1b_W, e1b_b, e1b_g, e1b_be, W2, b2, e2_W, e2_b, e2_g, e2_be, W3, b3):
    raise NotImplementedError("write your pallas kernel here")


# speedup vs baseline: 1.8994x; 1.8994x over previous
import functools

import jax
import jax.numpy as jnp
from jax import lax
from jax.experimental import pallas as pl
from jax.experimental.pallas import tpu as pltpu
from jax.experimental.pallas import tpu_sc as plsc

NN = 10000
NP = 10240
NE = 160000
EP = 163840
KNN = 20
NKP = NP * KNN
NREAL = NN * KNN
NC, NS = 2, 16
NW = NC * NS
CH = 128
INF = 3e38
BIGI = 2**30


def _sc_mesh():
    return plsc.VectorSubcoreMesh(core_axis_name="c", subcore_axis_name="s",
                                  num_cores=NC, num_subcores=NS)


def _hist_body(src_h, dst_h, ones_h, zeros_h, os_h, od_h,
               acc, ones_v, idx_v):
    c = lax.axis_index("c")
    s = lax.axis_index("s")
    rp = NP // NS
    pltpu.sync_copy(ones_h, ones_v)
    per_sc = EP // NC
    per_sub = per_sc // NS

    for idx_h, o_h in ((src_h, os_h), (dst_h, od_h)):
        pltpu.sync_copy(zeros_h.at[pl.ds(s * rp, rp)],
                        acc.at[pl.ds(s * rp, rp)])
        plsc.subcore_barrier()

        @pl.loop(0, per_sub // CH)
        def _(k, idx_h=idx_h):
            base = c * per_sc + s * per_sub + k * CH
            pltpu.sync_copy(idx_h.at[pl.ds(base, CH)], idx_v)
            pltpu.sync_copy(ones_v, acc.at[idx_v], add=True)

        plsc.subcore_barrier()
        pltpu.sync_copy(acc.at[pl.ds(s * rp, rp)],
                        o_h.at[c, pl.ds(s * rp, rp)])
        plsc.subcore_barrier()


def _sc_histogram(srcp, dstp, ones_rows, zeros128):
    out = jax.ShapeDtypeStruct((NC, NP, 128), jnp.float32)
    fn = pl.kernel(
        _hist_body,
        out_type=[out, out],
        mesh=_sc_mesh(),
        scratch_types=[
            pltpu.VMEM_SHARED((NP, 128), jnp.float32),
            pltpu.VMEM((CH, 128), jnp.float32),
            pltpu.VMEM((CH,), jnp.int32),
        ],
    )
    return fn(srcp, dstp, ones_rows, zeros128)


def _segsum_body(d, table_h, src_h, dst_h, zeros_h, out_h,
                 acc, src_v, dst_v, rows_v, sem):
    c = lax.axis_index("c")
    s = lax.axis_index("s")
    rp = NP // NS
    pltpu.sync_copy(zeros_h.at[pl.ds(s * rp, rp)], acc.at[pl.ds(s * rp, rp)])
    plsc.subcore_barrier()
    per_sc = EP // NC
    per_sub = per_sc // NS

    @pl.loop(0, per_sub // CH)
    def _(k):
        base = c * per_sc + s * per_sub + k * CH
        pltpu.sync_copy(src_h.at[pl.ds(base, CH)], src_v)
        pltpu.sync_copy(dst_h.at[pl.ds(base, CH)], dst_v)
        pltpu.async_copy(table_h.at[src_v], rows_v, sem).wait()
        pltpu.sync_copy(rows_v, acc.at[dst_v], add=True)

    plsc.subcore_barrier()
    pltpu.sync_copy(acc.at[pl.ds(s * rp, rp)], out_h.at[c, pl.ds(s * rp, rp)])


def _sc_segsum(table, srcp, dstp, zeros_d):
    d = table.shape[1]
    fn = pl.kernel(
        functools.partial(_segsum_body, d),
        out_type=jax.ShapeDtypeStruct((NC, NP, d), jnp.float32),
        mesh=_sc_mesh(),
        scratch_types=[
            pltpu.VMEM_SHARED((NP, d), jnp.float32),
            pltpu.VMEM((CH,), jnp.int32),
            pltpu.VMEM((CH,), jnp.int32),
            pltpu.VMEM((CH, d), jnp.float32),
            pltpu.SemaphoreType.DMA,
        ],
    )
    return fn(table, srcp, dstp, zeros_d)


def _gather_body(table_h, idx_h, out_h, idx_v, rows_v, sem):
    c = lax.axis_index("c")
    s = lax.axis_index("s")
    wid = s * NC + c
    per_w = NKP // NW

    @pl.loop(0, per_w // CH)
    def _(k):
        base = wid * per_w + k * CH
        pltpu.sync_copy(idx_h.at[pl.ds(base, CH)], idx_v)
        pltpu.async_copy(table_h.at[idx_v], rows_v, sem).wait()
        pltpu.sync_copy(rows_v, out_h.at[pl.ds(base, CH)])


def _sc_gather(table, idx_flat):
    d = table.shape[1]
    fn = pl.kernel(
        _gather_body,
        out_type=jax.ShapeDtypeStruct((NKP, d), jnp.float32),
        mesh=_sc_mesh(),
        scratch_types=[
            pltpu.VMEM((CH,), jnp.int32),
            pltpu.VMEM((CH, d), jnp.float32),
            pltpu.SemaphoreType.DMA,
        ],
    )
    return fn(table, idx_flat)


def _cnt_scale(cnt_blk):
    cnt = jnp.sum(cnt_blk, axis=0)[:, 0:1]
    return lax.rsqrt(jnp.maximum(cnt, 1.0))


def _mm_plain_body(x_ref, w_ref, o_ref):
    o_ref[...] = jnp.dot(x_ref[...], w_ref[...],
                         preferred_element_type=jnp.float32)


def _tc_mm_plain(x, w):
    din, dout = w.shape
    return pl.pallas_call(
        _mm_plain_body,
        grid=(NP // 128,),
        in_specs=[
            pl.BlockSpec((128, din), lambda i: (i, 0)),
            pl.BlockSpec((din, dout), lambda i: (0, 0)),
        ],
        out_specs=pl.BlockSpec((128, dout), lambda i: (i, 0)),
        out_shape=jax.ShapeDtypeStruct((NP, dout), jnp.float32),
    )(x, w)


def _mm_hs128_body(x_ref, w_ref, cnt_ref, o_ref):
    x = x_ref[...]
    scale = _cnt_scale(cnt_ref[...])
    o_ref[...] = jnp.dot(x, w_ref[...], preferred_element_type=jnp.float32) * scale


def _tc_mm_hs128(x, w128, cnt):
    return pl.pallas_call(
        _mm_hs128_body,
        grid=(NP // 128,),
        in_specs=[
            pl.BlockSpec((128, 256), lambda i: (i, 0)),
            pl.BlockSpec((256, 128), lambda i: (0, 0)),
            pl.BlockSpec((NC, 128, 128), lambda i: (0, i, 0)),
        ],
        out_specs=pl.BlockSpec((128, 128), lambda i: (i, 0)),
        out_shape=jax.ShapeDtypeStruct((NP, 128), jnp.float32),
    )(x, w128, cnt)


def _fin16_body(pa_ref, cnt_ref, b_ref, o_ref):
    pa = pa_ref[...]
    h = (pa[0] + pa[1])[:, :16]
    scale = _cnt_scale(cnt_ref[...])
    o_ref[...] = h * scale + b_ref[...]


FB16 = 1000


def _tc_finalize16(pa, cnt, b2d):
    return pl.pallas_call(
        _fin16_body,
        grid=(NN // FB16,),
        in_specs=[
            pl.BlockSpec((NC, FB16, 128), lambda i: (0, i, 0)),
            pl.BlockSpec((NC, FB16, 128), lambda i: (0, i, 0)),
            pl.BlockSpec((1, 16), lambda i: (0, 0)),
        ],
        out_specs=pl.BlockSpec((FB16, 16), lambda i: (i, 0)),
        out_shape=jax.ShapeDtypeStruct((NN, 16), jnp.float32),
    )(pa, cnt, b2d)


TI = 1024
NT = NP // TI


def _knn_body(ht_ref, sq_ref, sqr_ref, x_ref, o_ref, sc_ref):
    i = pl.program_id(0)
    x = x_ref[...]
    row0 = i * 128

    def p0(t, carry):
        d = jnp.dot(x, ht_ref[:, pl.ds(t * TI, TI)],
                    preferred_element_type=jnp.float32)
        s = sq_ref[:, pl.ds(t * TI, TI)]
        col = lax.broadcasted_iota(jnp.int32, (128, TI), 1) + t * TI
        rowi = lax.broadcasted_iota(jnp.int32, (128, TI), 0) + row0
        m = (col == rowi) | (col >= NN)
        sc_ref[:, pl.ds(t * TI, TI)] = jnp.where(
            m, INF, (sqr_ref[...] + s) - 2.0 * d)
        return carry
    lax.fori_loop(0, NT, p0, 0)

    cols = []
    for _k in range(KNN):
        def p1(t, m):
            blk = sc_ref[:, pl.ds(t * TI, TI)]
            return jnp.minimum(m, jnp.min(blk, axis=1, keepdims=True))
        mv = lax.fori_loop(0, NT, p1, jnp.full((128, 1), INF, jnp.float32))

        def p2(t, am):
            blk = sc_ref[:, pl.ds(t * TI, TI)]
            col = lax.broadcasted_iota(jnp.int32, (128, TI), 1) + t * TI
            cand = jnp.min(jnp.where(blk == mv, col, BIGI), axis=1,
                           keepdims=True)
            return jnp.minimum(am, cand)
        am = lax.fori_loop(0, NT, p2, jnp.full((128, 1), BIGI, jnp.int32))

        def p3(t, carry):
            blk = sc_ref[:, pl.ds(t * TI, TI)]
            col = lax.broadcasted_iota(jnp.int32, (128, TI), 1) + t * TI
            sc_ref[:, pl.ds(t * TI, TI)] = jnp.where(col == am, INF, blk)
            return carry
        lax.fori_loop(0, NT, p3, 0)
        cols.append(jnp.minimum(am, NN - 1))

    pad = jnp.zeros((128, 128 - KNN), jnp.int32)
    o_ref[...] = jnp.concatenate(cols + [pad], axis=1)


def _tc_knn(ht, sq2d, sqc, h):
    return pl.pallas_call(
        _knn_body,
        grid=(NP // 128,),
        in_specs=[
            pl.BlockSpec((256, NP), lambda i: (0, 0)),
            pl.BlockSpec((1, NP), lambda i: (0, 0)),
            pl.BlockSpec((128, 1), lambda i: (i, 0)),
            pl.BlockSpec((128, 256), lambda i: (i, 0)),
        ],
        out_specs=pl.BlockSpec((128, 128), lambda i: (i, 0)),
        out_shape=jax.ShapeDtypeStruct((NP, 128), jnp.int32),
        scratch_shapes=[pltpu.VMEM((128, NP), jnp.float32)],
    )(ht, sq2d, sqc, h)


NB = 40
RB = NB * KNN
GRID_E = NREAL // RB


def _bn_apply(y, ss):
    return jnp.maximum(((y - ss[0:1, :]) * ss[1:2, :]) * ss[2:3, :]
                       + ss[3:4, :], 0.0)


def _edge_y(g_blk, x_blk, w, b):
    xi = jnp.broadcast_to(x_blk[:, None, :], (NB, KNN, 256)).reshape(RB, 256)
    e = jnp.concatenate([xi, g_blk - xi], axis=1)
    return jnp.dot(e, w, preferred_element_type=jnp.float32) + b


def _edgey_body(g_ref, x_ref, w_ref, b_ref, y_ref):
    y_ref[...] = _edge_y(g_ref[...], x_ref[...], w_ref[...], b_ref[...])


def _tc_edge_y(g, x, w, b2d):
    return pl.pallas_call(
        _edgey_body,
        grid=(GRID_E,),
        in_specs=[
            pl.BlockSpec((RB, 256), lambda i: (i, 0)),
            pl.BlockSpec((NB, 256), lambda i: (i, 0)),
            pl.BlockSpec((512, 256), lambda i: (0, 0)),
            pl.BlockSpec((1, 256), lambda i: (0, 0)),
        ],
        out_specs=pl.BlockSpec((RB, 256), lambda i: (i, 0)),
        out_shape=jax.ShapeDtypeStruct((NREAL, 256), jnp.float32),
    )(g, x, w, b2d)


def _apply_mm_body(y_ref, ss_ref, w2_ref, b2_ref, v_ref):
    u = _bn_apply(y_ref[...], ss_ref[...])
    v_ref[...] = (jnp.dot(u, w2_ref[...], preferred_element_type=jnp.float32)
                  + b2_ref[...])


def _tc_apply_mm(y, ss, w2, b2d2):
    return pl.pallas_call(
        _apply_mm_body,
        grid=(GRID_E,),
        in_specs=[
            pl.BlockSpec((RB, 256), lambda i: (i, 0)),
            pl.BlockSpec((8, 256), lambda i: (0, 0)),
            pl.BlockSpec((256, 256), lambda i: (0, 0)),
            pl.BlockSpec((1, 256), lambda i: (0, 0)),
        ],
        out_specs=pl.BlockSpec((RB, 256), lambda i: (i, 0)),
        out_shape=jax.ShapeDtypeStruct((NREAL, 256), jnp.float32),
    )(y, ss, w2, b2d2)


def _bnmax_body(v_ref, ss_ref, o_ref):
    y = _bn_apply(v_ref[...], ss_ref[...])
    o_ref[...] = jnp.max(y.reshape(NB, KNN, 256), axis=1)


def _tc_bnmax(v, ss):
    return pl.pallas_call(
        _bnmax_body,
        grid=(GRID_E,),
        in_specs=[
            pl.BlockSpec((RB, 256), lambda i: (i, 0)),
            pl.BlockSpec((8, 256), lambda i: (0, 0)),
        ],
        out_specs=pl.BlockSpec((NB, 256), lambda i: (i, 0)),
        out_shape=jax.ShapeDtypeStruct((NP, 256), jnp.float32),
    )(v, ss)


def _applymax_body(y_ref, ss_ref, o_ref):
    u = _bn_apply(y_ref[...], ss_ref[...])
    o_ref[...] = jnp.max(u.reshape(NB, KNN, 256), axis=1)


def _tc_applymax(y, ss):
    return pl.pallas_call(
        _applymax_body,
        grid=(GRID_E,),
        in_specs=[
            pl.BlockSpec((RB, 256), lambda i: (i, 0)),
            pl.BlockSpec((8, 256), lambda i: (0, 0)),
        ],
        out_specs=pl.BlockSpec((NB, 256), lambda i: (i, 0)),
        out_shape=jax.ShapeDtypeStruct((NP, 256), jnp.float32),
    )(y, ss)


def _bn_rows(y, g, be):
    mu = jnp.mean(y, axis=0)
    var = jnp.var(y, axis=0)
    r = lax.rsqrt(var + 1e-5)
    pad = jnp.zeros((4, 256), jnp.float32)
    return jnp.concatenate([mu[None], r[None], g[None], be[None], pad], axis=0)


def _graph_conv(xpad, src, dst, w, b, deg_o, deg_i):
    h = _tc_mm_plain(xpad, w)[:NN]
    norm = (deg_o[src] ** -0.5) * (deg_i[dst] ** -0.5)
    msg = h[src] * norm[:, None]
    agg = jax.ops.segment_sum(msg, dst, num_segments=NN)
    out = jax.nn.relu(agg + b)
    return jnp.pad(out, ((0, NP - NN), (0, 0)))


def _dyn_edge(h, ew, eb, g, be):
    sq = jnp.sum(h * h, axis=1)
    idx = _tc_knn(jnp.transpose(h), sq.reshape(1, NP), sq.reshape(NP, 1), h)
    idx_flat = idx[:, :KNN].reshape(NKP)
    gt = _sc_gather(h, idx_flat)
    y = _tc_edge_y(gt, h, ew, eb.reshape(1, 256))
    ss = _bn_rows(y, g, be)
    return y, ss


def kernel(features, edge_index, W1, b1, e1a_W, e1a_b, e1a_g, e1a_be,
           e1b_W, e1b_b, e1b_g, e1b_be, W2, b2, e2_W, e2_b, e2_g, e2_be,
           W3, b3):
    src = edge_index[0].astype(jnp.int32)
    dst = edge_index[1].astype(jnp.int32)
    padv = jnp.full((EP - NE,), NP - 1, jnp.int32)
    srcp = jnp.concatenate([src, padv])
    dstp = jnp.concatenate([dst, padv])
    xpad = jnp.pad(features, ((0, NP - NN), (0, 0)))

    zeros128 = jnp.zeros((NP, 128), jnp.float32)
    ones_rows = jnp.zeros((CH, 128), jnp.float32).at[:, 0].set(1.0)

    co, cd = _sc_histogram(srcp, dstp, ones_rows, zeros128)
    deg_o = jnp.clip((co[0] + co[1])[:NN, 0], 1.0)
    deg_i = jnp.clip((cd[0] + cd[1])[:NN, 0], 1.0)

    h1 = _graph_conv(xpad, src, dst, W1, b1, deg_o, deg_i)

    y1, ss1 = _dyn_edge(h1, e1a_W, e1a_b, e1a_g, e1a_be)
    v1 = _tc_apply_mm(y1, ss1, e1b_W, e1b_b.reshape(1, 256))
    ss2 = _bn_rows(v1, e1b_g, e1b_be)
    h2 = _tc_bnmax(v1, ss2)

    h3 = _graph_conv(h2, src, dst, W2, b2, deg_o, deg_i)

    y2, ss3 = _dyn_edge(h3, e2_W, e2_b, e2_g, e2_be)
    h4 = _tc_applymax(y2, ss3)

    w3p = jnp.pad(W3, ((0, 0), (0, 128 - 16)))
    hs3 = _tc_mm_hs128(h4, w3p, co)
    p3 = _sc_segsum(hs3, srcp, dstp, zeros128)
    return _tc_finalize16(p3, cd, b3.reshape(1, 16))
